# Initial kernel scaffold; baseline (speedup 1.0000x reference)
import jax
import jax.numpy as jnp
from jax.experimental import pallas as pl


def kernel(x, H):
    raise NotImplementedError("write your pallas kernel here")


# speedup vs baseline: 3.9551x; 3.9551x over previous
import jax
import jax.numpy as jnp
from jax.experimental import pallas as pl
from jax.experimental.pallas import tpu as pltpu

_S = 64
_ROW = 256
_KW = 512


def _pqmf_mm_kernel(y_ref, yh_ref, w_ref, o_ref):
    X = y_ref[...]
    Xn = jnp.concatenate([X[1:], yh_ref[:1]], axis=0)
    Yw = jnp.concatenate([X, Xn], axis=1)
    R = jnp.dot(Yw, w_ref[...], preferred_element_type=jnp.float32)
    for k in range(o_ref.shape[0]):
        o_ref[k] = R[:, k * _S:(k + 1) * _S]


def kernel(x, H):
    B, C, T = x.shape
    Nb, taps1 = H.shape
    Tq = T // Nb
    pad = (taps1 - 1) // 2

    G = -(-Tq // _S)
    Gt = G
    for cand in (512, 256, 128, 64, 32, 16, 8):
        if G % cand == 0:
            Gt = cand
            break
    n_t = G // Gt
    rows = G + 8

    y = jnp.pad(x[:, 0, :], ((0, 0), (pad, rows * _ROW - T - pad)))
    y2 = y.reshape(B, rows, _ROW).astype(jnp.bfloat16)

    c = jnp.arange(_KW)[:, None]
    s = jnp.arange(_S)[None, :]
    d = c - 4 * s
    valid = (d >= 0) & (d < taps1)
    Hg = H.astype(jnp.float32)[:, jnp.clip(d, 0, taps1 - 1)]
    Wb = (jnp.where(valid[None], Hg, 0.0)
             .transpose(1, 0, 2)
             .reshape(_KW, Nb * _S)
             .astype(jnp.bfloat16))

    flops = 2 * B * G * _KW * (Nb * _S)
    bytes_accessed = 2 * B * rows * _ROW + 4 * B * Nb * G * _S

    out = pl.pallas_call(
        _pqmf_mm_kernel,
        out_shape=jax.ShapeDtypeStruct((B, Nb, G, _S), jnp.float32),
        grid=(B, n_t),
        in_specs=[
            pl.BlockSpec((None, Gt, _ROW), lambda b, i: (b, i, 0)),
            pl.BlockSpec((None, 8, _ROW), lambda b, i: (b, (i + 1) * (Gt // 8), 0)),
            pl.BlockSpec((_KW, Nb * _S), lambda b, i: (0, 0)),
        ],
        out_specs=pl.BlockSpec((None, Nb, Gt, _S), lambda b, i: (b, 0, i, 0)),
        compiler_params=pltpu.CompilerParams(
            dimension_semantics=("parallel", "parallel")),
        cost_estimate=pl.CostEstimate(flops=int(flops), transcendentals=0,
                                      bytes_accessed=int(bytes_accessed)),
    )(y2, y2, Wb)

    return out.reshape(B, Nb, G * _S)[:, :, :Tq]
